# baseline (speedup 1.0000x reference)
import functools

import jax
import jax.numpy as jnp
from jax import lax
from jax.experimental import pallas as pl
from jax.experimental.pallas import tpu as pltpu
from jax.experimental.pallas import tpu_sc as plsc

NC = 2
NS = 16
NW = NC * NS
CHUNK = 128
OPS_PER_STEP = 8
STEP_ROWS = CHUNK * OPS_PER_STEP
PACK = 4


def _gather_body(n_steps, table_hbm, idx_hbm, out_hbm, idx_v, rows_v, sem):
    wid = lax.axis_index("s") * NC + lax.axis_index("c")
    pltpu.sync_copy(idx_hbm.at[wid], idx_v)
    rows_per_w = n_steps * STEP_ROWS

    def step(i, carry):
        copies = []
        for k in range(OPS_PER_STEP):
            j = i * OPS_PER_STEP + k
            copies.append(
                pltpu.async_copy(
                    table_hbm.at[idx_v.at[j]],
                    rows_v.at[pl.ds(k * CHUNK, CHUNK)],
                    sem,
                )
            )
        for c in copies:
            c.wait()
        pltpu.sync_copy(
            rows_v,
            out_hbm.at[pl.ds(wid * rows_per_w + i * STEP_ROWS, STEP_ROWS)],
        )
        return carry

    lax.fori_loop(0, n_steps, step, 0)


def _sc_gather(table, idx3d):
    nw, n_chunks, chunk = idx3d.shape
    assert nw == NW and chunk == CHUNK and n_chunks % OPS_PER_STEP == 0
    n_steps = n_chunks // OPS_PER_STEP
    r = NW * n_chunks * CHUNK
    h = table.shape[1]
    mesh = plsc.VectorSubcoreMesh(core_axis_name="c", subcore_axis_name="s")
    return pl.kernel(
        functools.partial(_gather_body, n_steps),
        out_type=jax.ShapeDtypeStruct((r, h), jnp.float32),
        mesh=mesh,
        scratch_types=[
            pltpu.VMEM((n_chunks, CHUNK), jnp.int32),
            pltpu.VMEM((STEP_ROWS, h), jnp.float32),
            pltpu.SemaphoreType.DMA,
        ],
        compiler_params=pltpu.CompilerParams(use_tc_tiling_on_sc=False),
    )(table, idx3d)


def _repack_body(hmt_ref, o_ref):
    x = hmt_ref[...]
    xt = x.T
    h = x.shape[0]
    x4 = xt.reshape(xt.shape[0] // PACK, PACK, h)
    for u in range(PACK):
        o_ref[:, u * h:(u + 1) * h] = x4[:, u, :]


def _tc_repack(hmt, blk):
    h, v = hmt.shape
    grid = (pl.cdiv(v, blk),)
    return pl.pallas_call(
        _repack_body,
        grid=grid,
        in_specs=[pl.BlockSpec((h, blk), lambda i: (0, i))],
        out_specs=pl.BlockSpec((blk // PACK, PACK * h), lambda i: (i, 0)),
        out_shape=jax.ShapeDtypeStruct((v // PACK, PACK * h), jnp.float32),
    )(hmt)


def _proj_body(sblk, qb, g_ref, pm_ref, w_ref, o_ref):
    h = w_ref.shape[1] - pm_ref.shape[1]
    e = w_ref.shape[0]
    wh = w_ref[:, :h]
    wp = w_ref[:, h:]
    pospt = lax.dot_general(
        wp, pm_ref[...], (((1,), (1,)), ((), ())),
        preferred_element_type=jnp.float32, precision=lax.Precision.HIGHEST,
    )
    pcol = pospt.T[:, :, None]
    for u in range(PACK):
        g_u = g_ref[:, u * h:(u + 1) * h]
        o = lax.dot_general(
            wh, g_u, (((1,), (1,)), ((), ())),
            preferred_element_type=jnp.float32,
            precision=lax.Precision.HIGHEST,
        )
        o3 = o.reshape(e, sblk, qb).transpose(1, 0, 2)
        o_ref[:, :, u * qb:(u + 1) * qb] = o3 + pcol


def _tc_project_t(g_packed, pmatrix, W, b, s, e, sblk):
    rp, lanes = g_packed.shape
    qb = b // PACK
    grid = (s // sblk,)
    return pl.pallas_call(
        functools.partial(_proj_body, sblk, qb),
        grid=grid,
        in_specs=[
            pl.BlockSpec((sblk * qb, lanes), lambda i: (i, 0)),
            pl.BlockSpec((sblk, pmatrix.shape[1]), lambda i: (i, 0)),
            pl.BlockSpec(W.shape, lambda i: (0, 0)),
        ],
        out_specs=pl.BlockSpec((sblk, e, b), lambda i: (i, 0, 0)),
        out_shape=jax.ShapeDtypeStruct((s, e, b), jnp.float32),
    )(g_packed, pmatrix, W)


def kernel(xss, hmatrix, pmatrix, W):
    b, s = xss.shape
    v, h = hmatrix.shape
    p = pmatrix.shape[1]
    e = W.shape[0]
    r = b * s

    qb = b // PACK
    idx_sq = xss.T.reshape(s, PACK, qb).transpose(0, 2, 1)
    idx3d = idx_sq.reshape(NW, r // (NW * CHUNK), CHUNK)

    table_packed = _tc_repack(hmatrix.T, 16384)
    table_lin = table_packed.reshape(v, h)
    gathered = _sc_gather(table_lin, idx3d)

    g_packed = gathered.reshape(r // PACK, PACK * h)
    out_t = _tc_project_t(g_packed, pmatrix, W, b, s, e, 8)
    return out_t.transpose(2, 0, 1)


# speedup vs baseline: 1.5030x; 1.5030x over previous
import functools

import jax
import jax.numpy as jnp
from jax import lax
from jax.experimental import pallas as pl
from jax.experimental.pallas import tpu as pltpu
from jax.experimental.pallas import tpu_sc as plsc

NC = 2
NS = 16
NW = NC * NS
CHUNK = 128
OPS_PER_STEP = 8
STEP_ROWS = CHUNK * OPS_PER_STEP
PACK = 4


def _gather_body(n_steps, table_hbm, idx_hbm, out_hbm, idx_v, rows_v, sem):
    wid = lax.axis_index("s") * NC + lax.axis_index("c")
    pltpu.sync_copy(idx_hbm.at[wid], idx_v)
    rows_per_w = n_steps * STEP_ROWS

    def step(i, carry):
        copies = []
        for k in range(OPS_PER_STEP):
            j = i * OPS_PER_STEP + k
            copies.append(
                pltpu.async_copy(
                    table_hbm.at[idx_v.at[j]],
                    rows_v.at[pl.ds(k * CHUNK, CHUNK)],
                    sem,
                )
            )
        for c in copies:
            c.wait()
        pltpu.sync_copy(
            rows_v,
            out_hbm.at[pl.ds(wid * rows_per_w + i * STEP_ROWS, STEP_ROWS)],
        )
        return carry

    lax.fori_loop(0, n_steps, step, 0)


def _sc_gather(table, idx3d):
    nw, n_chunks, chunk = idx3d.shape
    assert nw == NW and chunk == CHUNK and n_chunks % OPS_PER_STEP == 0
    n_steps = n_chunks // OPS_PER_STEP
    r = NW * n_chunks * CHUNK
    h = table.shape[1]
    mesh = plsc.VectorSubcoreMesh(core_axis_name="c", subcore_axis_name="s")
    return pl.kernel(
        functools.partial(_gather_body, n_steps),
        out_type=jax.ShapeDtypeStruct((r, h), jnp.float32),
        mesh=mesh,
        scratch_types=[
            pltpu.VMEM((n_chunks, CHUNK), jnp.int32),
            pltpu.VMEM((STEP_ROWS, h), jnp.float32),
            pltpu.SemaphoreType.DMA,
        ],
        compiler_params=pltpu.CompilerParams(use_tc_tiling_on_sc=False),
    )(table, idx3d)


def _repack_body(hmt_ref, o_ref):
    x = hmt_ref[...]
    xt = x.T
    h = x.shape[0]
    x4 = xt.reshape(xt.shape[0] // PACK, PACK, h)
    for u in range(PACK):
        o_ref[:, u * h:(u + 1) * h] = x4[:, u, :]


def _tc_repack(hmt, blk):
    h, v = hmt.shape
    grid = (pl.cdiv(v, blk),)
    return pl.pallas_call(
        _repack_body,
        grid=grid,
        in_specs=[pl.BlockSpec((h, blk), lambda i: (0, i))],
        out_specs=pl.BlockSpec((blk // PACK, PACK * h), lambda i: (i, 0)),
        out_shape=jax.ShapeDtypeStruct((v // PACK, PACK * h), jnp.float32),
    )(hmt)


def _proj_body(sblk, qb, g_ref, pm_ref, w_ref, o_ref):
    h = w_ref.shape[1] - pm_ref.shape[1]
    e = w_ref.shape[0]
    wh = w_ref[:, :h]
    wp = w_ref[:, h:]
    pospt = lax.dot_general(
        wp, pm_ref[...], (((1,), (1,)), ((), ())),
        preferred_element_type=jnp.float32, precision=lax.Precision.HIGHEST,
    )
    pcol = pospt.T[:, :, None]
    for u in range(PACK):
        g_u = g_ref[:, u * h:(u + 1) * h]
        o = lax.dot_general(
            wh, g_u, (((1,), (1,)), ((), ())),
            preferred_element_type=jnp.float32,
            precision=lax.Precision.DEFAULT,
        )
        o3 = o.reshape(e, sblk, qb).transpose(1, 0, 2)
        o_ref[:, :, u * qb:(u + 1) * qb] = o3 + pcol


def _tc_project_t(g_packed, pmatrix, W, b, s, e, sblk):
    rp, lanes = g_packed.shape
    qb = b // PACK
    grid = (s // sblk,)
    return pl.pallas_call(
        functools.partial(_proj_body, sblk, qb),
        grid=grid,
        in_specs=[
            pl.BlockSpec((sblk * qb, lanes), lambda i: (i, 0)),
            pl.BlockSpec((sblk, pmatrix.shape[1]), lambda i: (i, 0)),
            pl.BlockSpec(W.shape, lambda i: (0, 0)),
        ],
        out_specs=pl.BlockSpec((sblk, e, b), lambda i: (i, 0, 0)),
        out_shape=jax.ShapeDtypeStruct((s, e, b), jnp.float32),
    )(g_packed, pmatrix, W)


def kernel(xss, hmatrix, pmatrix, W):
    b, s = xss.shape
    v, h = hmatrix.shape
    p = pmatrix.shape[1]
    e = W.shape[0]
    r = b * s

    qb = b // PACK
    idx_sq = xss.T.reshape(s, PACK, qb).transpose(0, 2, 1)
    idx3d = idx_sq.reshape(NW, r // (NW * CHUNK), CHUNK)

    table_packed = _tc_repack(hmatrix.T, 16384)
    table_lin = table_packed.reshape(v, h)
    gathered = _sc_gather(table_lin, idx3d)

    g_packed = gathered.reshape(r // PACK, PACK * h)
    out_t = _tc_project_t(g_packed, pmatrix, W, b, s, e, 8)
    return out_t.transpose(2, 0, 1)


# speedup vs baseline: 1.5133x; 1.0068x over previous
import functools

import jax
import jax.numpy as jnp
from jax import lax
from jax.experimental import pallas as pl
from jax.experimental.pallas import tpu as pltpu
from jax.experimental.pallas import tpu_sc as plsc

NC = 2
NS = 16
NW = NC * NS
CHUNK = 128
OPS_PER_STEP = 8
STEP_ROWS = CHUNK * OPS_PER_STEP
PACK = 4


def _gather_body(n_steps, table_hbm, idx_hbm, out_hbm, idx_v, rows_v, sem):
    wid = lax.axis_index("s") * NC + lax.axis_index("c")
    pltpu.sync_copy(idx_hbm.at[wid], idx_v)
    rows_per_w = n_steps * STEP_ROWS

    def step(i, carry):
        copies = []
        for k in range(OPS_PER_STEP):
            j = i * OPS_PER_STEP + k
            copies.append(
                pltpu.async_copy(
                    table_hbm.at[idx_v.at[j]],
                    rows_v.at[pl.ds(k * CHUNK, CHUNK)],
                    sem,
                )
            )
        for c in copies:
            c.wait()
        pltpu.sync_copy(
            rows_v,
            out_hbm.at[pl.ds(wid * rows_per_w + i * STEP_ROWS, STEP_ROWS)],
        )
        return carry

    lax.fori_loop(0, n_steps, step, 0)


def _sc_gather(table, idx3d):
    nw, n_chunks, chunk = idx3d.shape
    assert nw == NW and chunk == CHUNK and n_chunks % OPS_PER_STEP == 0
    n_steps = n_chunks // OPS_PER_STEP
    r = NW * n_chunks * CHUNK
    h = table.shape[1]
    mesh = plsc.VectorSubcoreMesh(core_axis_name="c", subcore_axis_name="s")
    return pl.kernel(
        functools.partial(_gather_body, n_steps),
        out_type=jax.ShapeDtypeStruct((r, h), jnp.float32),
        mesh=mesh,
        scratch_types=[
            pltpu.VMEM((n_chunks, CHUNK), jnp.int32),
            pltpu.VMEM((STEP_ROWS, h), jnp.float32),
            pltpu.SemaphoreType.DMA,
        ],
        compiler_params=pltpu.CompilerParams(use_tc_tiling_on_sc=False),
    )(table, idx3d)


def _repack_body(hmt_ref, o_ref):
    x = hmt_ref[...]
    xt = x.T
    h = x.shape[0]
    x4 = xt.reshape(xt.shape[0] // PACK, PACK, h)
    for u in range(PACK):
        o_ref[:, u * h:(u + 1) * h] = x4[:, u, :]


def _tc_repack(hmt, blk):
    h, v = hmt.shape
    grid = (pl.cdiv(v, blk),)
    return pl.pallas_call(
        _repack_body,
        grid=grid,
        in_specs=[pl.BlockSpec((h, blk), lambda i: (0, i))],
        out_specs=pl.BlockSpec((blk // PACK, PACK * h), lambda i: (i, 0)),
        out_shape=jax.ShapeDtypeStruct((v // PACK, PACK * h), jnp.float32),
    )(hmt)


def _proj_body(sblk, qb, g_ref, pm_ref, w_ref, o_ref):
    h = w_ref.shape[1] - pm_ref.shape[1]
    e = w_ref.shape[0]
    wh = w_ref[:, :h]
    wp = w_ref[:, h:]
    pospt = lax.dot_general(
        wp, pm_ref[...], (((1,), (1,)), ((), ())),
        preferred_element_type=jnp.float32, precision=lax.Precision.HIGHEST,
    )
    pcol = pospt.T[:, :, None]
    for u in range(PACK):
        g_u = g_ref[:, u * h:(u + 1) * h]
        o = lax.dot_general(
            wh, g_u, (((1,), (1,)), ((), ())),
            preferred_element_type=jnp.float32,
            precision=lax.Precision.DEFAULT,
        )
        o3 = o.reshape(e, sblk, qb).transpose(1, 0, 2)
        o_ref[:, :, u * qb:(u + 1) * qb] = o3 + pcol


def _tc_project_t(g_packed, pmatrix, W, b, s, e, sblk):
    rp, lanes = g_packed.shape
    qb = b // PACK
    grid = (s // sblk,)
    return pl.pallas_call(
        functools.partial(_proj_body, sblk, qb),
        grid=grid,
        in_specs=[
            pl.BlockSpec((sblk * qb, lanes), lambda i: (i, 0)),
            pl.BlockSpec((sblk, pmatrix.shape[1]), lambda i: (i, 0)),
            pl.BlockSpec(W.shape, lambda i: (0, 0)),
        ],
        out_specs=pl.BlockSpec((sblk, e, b), lambda i: (i, 0, 0)),
        out_shape=jax.ShapeDtypeStruct((s, e, b), jnp.float32),
    )(g_packed, pmatrix, W)


def kernel(xss, hmatrix, pmatrix, W):
    b, s = xss.shape
    v, h = hmatrix.shape
    p = pmatrix.shape[1]
    e = W.shape[0]
    r = b * s

    qb = b // PACK
    idx_sq = xss.T.reshape(s, PACK, qb).transpose(0, 2, 1)
    idx3d = idx_sq.reshape(NW, r // (NW * CHUNK), CHUNK)

    table_packed = _tc_repack(hmatrix.T, 24576)
    table_lin = table_packed.reshape(v, h)
    gathered = _sc_gather(table_lin, idx3d)

    g_packed = gathered.reshape(r // PACK, PACK * h)
    out_t = _tc_project_t(g_packed, pmatrix, W, b, s, e, 8)
    return out_t.transpose(2, 0, 1)


# speedup vs baseline: 1.5139x; 1.0004x over previous
import functools

import jax
import jax.numpy as jnp
from jax import lax
from jax.experimental import pallas as pl
from jax.experimental.pallas import tpu as pltpu
from jax.experimental.pallas import tpu_sc as plsc

NC = 2
NS = 16
NW = NC * NS
CHUNK = 128
OPS_PER_STEP = 8
STEP_ROWS = CHUNK * OPS_PER_STEP
PACK = 4


def _gather_body(n_steps, table_hbm, idx_hbm, out_hbm, idx_v, rows_v, sem):
    wid = lax.axis_index("s") * NC + lax.axis_index("c")
    pltpu.sync_copy(idx_hbm.at[wid], idx_v)
    rows_per_w = n_steps * STEP_ROWS

    def step(i, carry):
        copies = []
        for k in range(OPS_PER_STEP):
            j = i * OPS_PER_STEP + k
            copies.append(
                pltpu.async_copy(
                    table_hbm.at[idx_v.at[j]],
                    rows_v.at[pl.ds(k * CHUNK, CHUNK)],
                    sem,
                )
            )
        for c in copies:
            c.wait()
        pltpu.sync_copy(
            rows_v,
            out_hbm.at[pl.ds(wid * rows_per_w + i * STEP_ROWS, STEP_ROWS)],
        )
        return carry

    lax.fori_loop(0, n_steps, step, 0)


def _sc_gather(table, idx3d):
    nw, n_chunks, chunk = idx3d.shape
    assert nw == NW and chunk == CHUNK and n_chunks % OPS_PER_STEP == 0
    n_steps = n_chunks // OPS_PER_STEP
    r = NW * n_chunks * CHUNK
    h = table.shape[1]
    mesh = plsc.VectorSubcoreMesh(core_axis_name="c", subcore_axis_name="s")
    return pl.kernel(
        functools.partial(_gather_body, n_steps),
        out_type=jax.ShapeDtypeStruct((r, h), jnp.float32),
        mesh=mesh,
        scratch_types=[
            pltpu.VMEM((n_chunks, CHUNK), jnp.int32),
            pltpu.VMEM((STEP_ROWS, h), jnp.float32),
            pltpu.SemaphoreType.DMA,
        ],
        compiler_params=pltpu.CompilerParams(use_tc_tiling_on_sc=False),
    )(table, idx3d)


def _repack_body(hmt_ref, o_ref):
    x = hmt_ref[...]
    xt = x.T
    h = x.shape[0]
    x4 = xt.reshape(xt.shape[0] // PACK, PACK, h)
    for u in range(PACK):
        o_ref[:, u * h:(u + 1) * h] = x4[:, u, :]


def _tc_repack(hmt, blk):
    h, v = hmt.shape
    grid = (pl.cdiv(v, blk),)
    return pl.pallas_call(
        _repack_body,
        grid=grid,
        in_specs=[pl.BlockSpec((h, blk), lambda i: (0, i))],
        out_specs=pl.BlockSpec((blk // PACK, PACK * h), lambda i: (i, 0)),
        out_shape=jax.ShapeDtypeStruct((v // PACK, PACK * h), jnp.float32),
    )(hmt)


def _proj_body(sblk, qb, g_ref, pm_ref, w_ref, o_ref):
    h = w_ref.shape[1] - pm_ref.shape[1]
    e = w_ref.shape[0]
    wh = w_ref[:, :h]
    wp = w_ref[:, h:]
    pospt = lax.dot_general(
        wp, pm_ref[...], (((1,), (1,)), ((), ())),
        preferred_element_type=jnp.float32, precision=lax.Precision.HIGHEST,
    )
    pcol = pospt.T[:, :, None]
    for u in range(PACK):
        g_u = g_ref[:, u * h:(u + 1) * h]
        o = lax.dot_general(
            wh, g_u, (((1,), (1,)), ((), ())),
            preferred_element_type=jnp.float32,
            precision=lax.Precision.DEFAULT,
        )
        o3 = o.reshape(e, sblk, qb).transpose(1, 0, 2)
        o_ref[:, :, u * qb:(u + 1) * qb] = o3 + pcol


def _tc_project_t(g_packed, pmatrix, W, b, s, e, sblk):
    rp, lanes = g_packed.shape
    qb = b // PACK
    grid = (s // sblk,)
    return pl.pallas_call(
        functools.partial(_proj_body, sblk, qb),
        grid=grid,
        in_specs=[
            pl.BlockSpec((sblk * qb, lanes), lambda i: (i, 0)),
            pl.BlockSpec((sblk, pmatrix.shape[1]), lambda i: (i, 0)),
            pl.BlockSpec(W.shape, lambda i: (0, 0)),
        ],
        out_specs=pl.BlockSpec((sblk, e, b), lambda i: (i, 0, 0)),
        out_shape=jax.ShapeDtypeStruct((s, e, b), jnp.float32),
    )(g_packed, pmatrix, W)


def kernel(xss, hmatrix, pmatrix, W):
    b, s = xss.shape
    v, h = hmatrix.shape
    p = pmatrix.shape[1]
    e = W.shape[0]
    r = b * s

    qb = b // PACK
    idx_sq = xss.reshape(PACK, qb, s).transpose(2, 1, 0)
    idx3d = idx_sq.reshape(NW, r // (NW * CHUNK), CHUNK)

    table_packed = _tc_repack(hmatrix.T, 24576)
    table_lin = table_packed.reshape(v, h)
    gathered = _sc_gather(table_lin, idx3d)

    g_packed = gathered.reshape(r // PACK, PACK * h)
    out_t = _tc_project_t(g_packed, pmatrix, W, b, s, e, 8)
    return out_t.transpose(2, 0, 1)
